# baseline (speedup 1.0000x reference)
import functools

import jax
import jax.numpy as jnp
from jax import lax
from jax.experimental import pallas as pl
from jax.experimental.pallas import tpu as pltpu
from jax.experimental.pallas import tpu_sc as plsc

B = 16384
D = 64
NW = 32
BPW = B // NW
C = 128
NCHUNK = BPW // C


def _sc_gather_body(uidx, midx, eu_mf, em_mf, eu_mlp, em_mlp,
                    o_umf, o_mmf, o_umlp, o_mmlp,
                    uiv, miv, bu_mf, bm_mf, bu_mlp, bm_mlp, sem):
    wid = lax.axis_index("s") * 2 + lax.axis_index("c")
    base = wid * BPW
    for c in range(NCHUNK):
        off = base + c * C
        pltpu.sync_copy(uidx.at[pl.ds(off, C)], uiv)
        pltpu.sync_copy(midx.at[pl.ds(off, C)], miv)
        d0 = pltpu.async_copy(eu_mf.at[uiv], bu_mf, sem)
        d1 = pltpu.async_copy(em_mf.at[miv], bm_mf, sem)
        d2 = pltpu.async_copy(eu_mlp.at[uiv], bu_mlp, sem)
        d3 = pltpu.async_copy(em_mlp.at[miv], bm_mlp, sem)
        d0.wait()
        d1.wait()
        d2.wait()
        d3.wait()
        pltpu.sync_copy(bu_mf, o_umf.at[pl.ds(off, C)])
        pltpu.sync_copy(bm_mf, o_mmf.at[pl.ds(off, C)])
        pltpu.sync_copy(bu_mlp, o_umlp.at[pl.ds(off, C)])
        pltpu.sync_copy(bm_mlp, o_mmlp.at[pl.ds(off, C)])


_row = jax.ShapeDtypeStruct((B, D), jnp.float32)
_sc_gather = functools.partial(
    pl.kernel,
    out_type=(_row, _row, _row, _row),
    mesh=plsc.VectorSubcoreMesh(core_axis_name="c", subcore_axis_name="s"),
    scratch_types=[
        pltpu.VMEM((C,), jnp.int32),
        pltpu.VMEM((C,), jnp.int32),
        pltpu.VMEM((C, D), jnp.float32),
        pltpu.VMEM((C, D), jnp.float32),
        pltpu.VMEM((C, D), jnp.float32),
        pltpu.VMEM((C, D), jnp.float32),
        pltpu.SemaphoreType.DMA,
    ],
    compiler_params=pltpu.CompilerParams(use_tc_tiling_on_sc=False),
)(_sc_gather_body)


BB = 1024
GRID = B // BB


def _tc_mlp_body(umf, mmf, umlp, mmlp, w1u, w1m, b1, w2, b2, w3, b3, w4, b4,
                 wf_mf, wf_h, bf, out):
    mf = umf[...] * mmf[...]
    h = jnp.maximum(
        jnp.dot(umlp[...], w1u[...], preferred_element_type=jnp.float32)
        + jnp.dot(mmlp[...], w1m[...], preferred_element_type=jnp.float32)
        + b1[...], 0.0)
    h = jnp.maximum(jnp.dot(h, w2[...], preferred_element_type=jnp.float32) + b2[...], 0.0)
    h = jnp.maximum(jnp.dot(h, w3[...], preferred_element_type=jnp.float32) + b3[...], 0.0)
    h = jnp.maximum(jnp.dot(h, w4[...], preferred_element_type=jnp.float32) + b4[...], 0.0)
    pred = (jnp.sum(mf * wf_mf[...], axis=-1)
            + jnp.sum(h * wf_h[...], axis=-1) + bf[0, 0])
    out[...] = jax.nn.sigmoid(pred)


def _const2d(shape):
    return pl.BlockSpec(shape, lambda i: (0, 0))


def kernel(user_indices, movie_indices, Eu_mf, Em_mf, Eu_mlp, Em_mlp,
           W1, b1, W2, b2, W3, b3, W4, b4, Wf, bf):
    ue_mf, me_mf, ue_mlp, me_mlp = _sc_gather(
        user_indices, movie_indices, Eu_mf, Em_mf, Eu_mlp, Em_mlp)

    row_spec = pl.BlockSpec((BB, D), lambda i: (i, 0))
    out = pl.pallas_call(
        _tc_mlp_body,
        grid=(GRID,),
        in_specs=[
            row_spec, row_spec, row_spec, row_spec,
            _const2d((D, 128)), _const2d((D, 128)), _const2d((1, 128)),
            _const2d((128, 64)), _const2d((1, 64)),
            _const2d((64, 32)), _const2d((1, 32)),
            _const2d((32, 16)), _const2d((1, 16)),
            _const2d((1, D)), _const2d((1, 16)), _const2d((1, 1)),
        ],
        out_specs=pl.BlockSpec((BB,), lambda i: (i,)),
        out_shape=jax.ShapeDtypeStruct((B,), jnp.float32),
        compiler_params=pltpu.CompilerParams(
            dimension_semantics=("arbitrary",),
        ),
    )(
        ue_mf, me_mf, ue_mlp, me_mlp,
        W1[:D], W1[D:], b1.reshape(1, 128),
        W2, b2.reshape(1, 64),
        W3, b3.reshape(1, 32),
        W4, b4.reshape(1, 16),
        Wf[:D, 0].reshape(1, D), Wf[D:, 0].reshape(1, 16), bf.reshape(1, 1),
    )
    return out
